# baseline (speedup 1.0000x reference)
import functools

import jax
import jax.numpy as jnp
from jax import lax
from jax.experimental import pallas as pl
from jax.experimental.pallas import tpu as pltpu
from jax.experimental.pallas import tpu_sc as plsc

_EPS = 1e-05
_IDX_CHUNK = 128
_CH = 128


def _sc_gather_flat(Eh_flat, rvh, Wu, bs, bo, u_idx, r_idx, v_idx, D):
    B = u_idx.shape[0]
    info = plsc.get_sparse_core_info()
    nw = info.num_cores * info.num_subcores
    bpw = B // nw
    groups = bpw // 16

    mesh = plsc.VectorSubcoreMesh(core_axis_name="c", subcore_axis_name="s")

    @functools.partial(
        pl.kernel,
        out_type=(
            jax.ShapeDtypeStruct((B * D,), jnp.float32),
            jax.ShapeDtypeStruct((B * D,), jnp.float32),
            jax.ShapeDtypeStruct((B, D), jnp.float32),
            jax.ShapeDtypeStruct((B, D), jnp.float32),
            jax.ShapeDtypeStruct((B,), jnp.float32),
            jax.ShapeDtypeStruct((B,), jnp.float32),
        ),
        mesh=mesh,
        compiler_params=pltpu.CompilerParams(use_tc_tiling_on_sc=True),
        scratch_types=[
            pltpu.VMEM((bpw,), jnp.int32),
            pltpu.VMEM((bpw,), jnp.int32),
            pltpu.VMEM((bpw,), jnp.int32),
            pltpu.VMEM((_CH * D,), jnp.float32),
            pltpu.VMEM((_CH * D,), jnp.float32),
            pltpu.VMEM((_CH, D), jnp.float32),
            pltpu.VMEM((_CH, D), jnp.float32),
            pltpu.VMEM((bpw,), jnp.float32),
            pltpu.VMEM((bpw,), jnp.float32),
            pltpu.SemaphoreType.DMA,
            pltpu.SemaphoreType.DMA,
        ],
    )
    def k(eh, rv, wu, bs_t, bo_t, uix, rix, vix,
          u_out, v_out, ru_out, rg_out, bsu_out, bov_out,
          uix_v, rix_v, vix_v, u_v, v_v, ru_v, rg_v, bsu_v, bov_v, sem, bsem):
        wid = lax.axis_index("s") * info.num_cores + lax.axis_index("c")
        base = wid * bpw
        pltpu.sync_copy(uix.at[pl.ds(base, bpw)], uix_v)
        pltpu.sync_copy(rix.at[pl.ds(base, bpw)], rix_v)
        pltpu.sync_copy(vix.at[pl.ds(base, bpw)], vix_v)

        bdescs = []
        for j in range(bpw // _IDX_CHUNK):
            sl = pl.ds(j * _IDX_CHUNK, _IDX_CHUNK)
            bdescs.append(pltpu.async_copy(bs_t.at[uix_v.at[sl]], bsu_v.at[sl], bsem))
            bdescs.append(pltpu.async_copy(bo_t.at[vix_v.at[sl]], bov_v.at[sl], bsem))

        def chunk(c, _):
            coff = pl.multiple_of(c * _CH, _CH)

            def grp(g, _):
                goff = pl.multiple_of(coff + g * 16, 16)
                boff = pl.multiple_of(g * 16, 16)
                iu = uix_v[pl.ds(goff, 16)]
                iv = vix_v[pl.ds(goff, 16)]
                ir = rix_v[pl.ds(goff, 16)]
                descs = []
                for j in range(16):
                    dst = pl.ds(boff + j, 1)
                    dstf = pl.ds((boff + j) * D, D)
                    descs.append(pltpu.async_copy(eh.at[pl.ds(iu[j] * D, D)], u_v.at[dstf], sem))
                    descs.append(pltpu.async_copy(eh.at[pl.ds(iv[j] * D, D)], v_v.at[dstf], sem))
                    descs.append(pltpu.async_copy(wu.at[pl.ds(ir[j], 1)], ru_v.at[dst], sem))
                    descs.append(pltpu.async_copy(rv.at[pl.ds(ir[j], 1)], rg_v.at[dst], sem))
                for d in descs:
                    d.wait()
                return ()

            lax.fori_loop(0, _CH // 16, grp, (), unroll=False)
            pltpu.sync_copy(u_v, u_out.at[pl.ds((base + coff) * D, _CH * D)])
            pltpu.sync_copy(v_v, v_out.at[pl.ds((base + coff) * D, _CH * D)])
            out_sl = pl.ds(base + coff, _CH)
            pltpu.sync_copy(ru_v, ru_out.at[out_sl])
            pltpu.sync_copy(rg_v, rg_out.at[out_sl])
            return ()

        lax.fori_loop(0, bpw // _CH, chunk, (), unroll=False)

        for d in bdescs:
            d.wait()

        out_sl = pl.ds(base, bpw)
        pltpu.sync_copy(bsu_v, bsu_out.at[out_sl])
        pltpu.sync_copy(bov_v, bov_out.at[out_sl])

    return k(Eh_flat, rvh, Wu, bs, bo, u_idx, r_idx, v_idx)


def _math_body(u_ref, v_ref, ru_ref, rg_ref, bsu_ref, bov_ref, o_ref):
    u = u_ref[...]
    v = v_ref[...]
    Ru = ru_ref[...]
    rg = rg_ref[...]

    def artanh(x):
        return 0.5 * jnp.log((1.0 + x) / (1.0 - x))

    def sqnorm(t):
        return jnp.sum(t * t, axis=-1, keepdims=True)

    def clip_ball(t):
        nrm = jnp.sqrt(sqnorm(t))
        return jnp.where(nrm >= 1.0, t / (nrm - _EPS), t)

    def p_sum(x, y):
        sqx = jnp.clip(sqnorm(x), 0.0, 1.0 - 1e-5)
        sqy = jnp.clip(sqnorm(y), 0.0, 1.0 - 1e-5)
        dotxy = jnp.sum(x * y, axis=-1, keepdims=True)
        num = (1.0 + 2.0 * dotxy + sqy) * x + (1.0 - sqx) * y
        den = 1.0 + 2.0 * dotxy + sqx * sqy
        return num / den

    u = clip_ball(u)
    v = clip_ball(v)
    rg = clip_ball(rg)
    nu = jnp.clip(jnp.sqrt(sqnorm(u)), 1e-10, 1.0 - 1e-5)
    u_w = (artanh(nu) / nu) * u * Ru
    nw = jnp.clip(jnp.sqrt(sqnorm(u_w)), 1e-10, None)
    u_m = (jnp.tanh(nw) / nw) * u_w
    v_m = p_sum(v, rg)
    u_m = clip_ball(u_m)
    v_m = clip_ball(v_m)
    dn = jnp.clip(jnp.sqrt(sqnorm(p_sum(-u_m, v_m))), 1e-10, 1.0 - 1e-5)
    sqdist = (2.0 * artanh(dn)) ** 2
    o_ref[...] = -sqdist + bsu_ref[...] + bov_ref[...]


def _tc_math(u, v, ru, rg, bsu, bov):
    B, D = u.shape
    blk = 2048
    grid = B // blk
    row_spec = pl.BlockSpec((blk, D), lambda i: (i, 0))
    col_spec = pl.BlockSpec((blk, 1), lambda i: (i, 0))
    out = pl.pallas_call(
        _math_body,
        grid=(grid,),
        in_specs=[row_spec, row_spec, row_spec, row_spec, col_spec, col_spec],
        out_specs=col_spec,
        out_shape=jax.ShapeDtypeStruct((B, 1), jnp.float32),
    )(u, v, ru, rg, bsu.reshape(B, 1), bov.reshape(B, 1))
    return out.reshape(B)


def kernel(u_idx, r_idx, v_idx, Eh, rvh, Wu, bs, bo):
    B = u_idx.shape[0]
    D = Eh.shape[1]
    u, v, ru, rg, bsu, bov = _sc_gather_flat(
        Eh.reshape(-1), rvh, Wu, bs, bo, u_idx, r_idx, v_idx, D)
    return (u.reshape(B, D), v.reshape(B, D), ru, rg, bsu, bov)


# speedup vs baseline: 1.0367x; 1.0367x over previous
import functools

import jax
import jax.numpy as jnp
from jax import lax
from jax.experimental import pallas as pl
from jax.experimental.pallas import tpu as pltpu
from jax.experimental.pallas import tpu_sc as plsc

_EPS = 1e-05
_IDX_CHUNK = 128


def _sc_gather(Eh, rvh, Wu, bs, bo, u_idx, r_idx, v_idx):
    B = u_idx.shape[0]
    D = Eh.shape[1]
    info = plsc.get_sparse_core_info()
    nw = info.num_cores * info.num_subcores
    bpw = B // nw

    mesh = plsc.VectorSubcoreMesh(core_axis_name="c", subcore_axis_name="s")

    @functools.partial(
        pl.kernel,
        out_type=(
            jax.ShapeDtypeStruct((B, D), jnp.float32),
            jax.ShapeDtypeStruct((B, D), jnp.float32),
            jax.ShapeDtypeStruct((B, D), jnp.float32),
            jax.ShapeDtypeStruct((B, D), jnp.float32),
            jax.ShapeDtypeStruct((B,), jnp.float32),
            jax.ShapeDtypeStruct((B,), jnp.float32),
        ),
        mesh=mesh,
        compiler_params=pltpu.CompilerParams(use_tc_tiling_on_sc=False),
        scratch_types=[
            pltpu.VMEM((bpw,), jnp.int32),
            pltpu.VMEM((bpw,), jnp.int32),
            pltpu.VMEM((bpw,), jnp.int32),
            pltpu.VMEM((bpw, D), jnp.float32),
            pltpu.VMEM((bpw, D), jnp.float32),
            pltpu.VMEM((bpw, D), jnp.float32),
            pltpu.VMEM((bpw, D), jnp.float32),
            pltpu.VMEM((bpw,), jnp.float32),
            pltpu.VMEM((bpw,), jnp.float32),
            pltpu.SemaphoreType.DMA,
        ],
    )
    def k(eh, rv, wu, bs_t, bo_t, uix, rix, vix,
          u_out, v_out, ru_out, rg_out, bsu_out, bov_out,
          uix_v, rix_v, vix_v, u_v, v_v, ru_v, rg_v, bsu_v, bov_v, sem):
        wid = lax.axis_index("s") * info.num_cores + lax.axis_index("c")
        base = wid * bpw
        pltpu.sync_copy(uix.at[pl.ds(base, bpw)], uix_v)
        pltpu.sync_copy(rix.at[pl.ds(base, bpw)], rix_v)
        pltpu.sync_copy(vix.at[pl.ds(base, bpw)], vix_v)
        descs = []
        for j in range(bpw // _IDX_CHUNK):
            sl = pl.ds(j * _IDX_CHUNK, _IDX_CHUNK)
            descs.append(pltpu.async_copy(eh.at[uix_v.at[sl]], u_v.at[sl], sem))
            descs.append(pltpu.async_copy(eh.at[vix_v.at[sl]], v_v.at[sl], sem))
            descs.append(pltpu.async_copy(wu.at[rix_v.at[sl]], ru_v.at[sl], sem))
            descs.append(pltpu.async_copy(rv.at[rix_v.at[sl]], rg_v.at[sl], sem))
            descs.append(pltpu.async_copy(bs_t.at[uix_v.at[sl]], bsu_v.at[sl], sem))
            descs.append(pltpu.async_copy(bo_t.at[vix_v.at[sl]], bov_v.at[sl], sem))
        for d in descs:
            d.wait()
        out_sl = pl.ds(base, bpw)
        pltpu.sync_copy(u_v, u_out.at[out_sl])
        pltpu.sync_copy(v_v, v_out.at[out_sl])
        pltpu.sync_copy(ru_v, ru_out.at[out_sl])
        pltpu.sync_copy(rg_v, rg_out.at[out_sl])
        pltpu.sync_copy(bsu_v, bsu_out.at[out_sl])
        pltpu.sync_copy(bov_v, bov_out.at[out_sl])

    return k(Eh, rvh, Wu, bs, bo, u_idx, r_idx, v_idx)


def kernel(u_idx, r_idx, v_idx, Eh, rvh, Wu, bs, bo):
    return _sc_gather(Eh, rvh, Wu, bs, bo, u_idx, r_idx, v_idx)


# speedup vs baseline: 1.5484x; 1.4936x over previous
import functools

import jax
import jax.numpy as jnp
from jax import lax
from jax.experimental import pallas as pl
from jax.experimental.pallas import tpu as pltpu
from jax.experimental.pallas import tpu_sc as plsc

_EPS = 1e-05
_IDX_CHUNK = 128
_CH = 128
_LN2 = 0.6931471805599453


def _rsqrt(x):
    bits = jnp.int32(0x5F3759DF) - lax.shift_right_logical(plsc.bitcast(x, jnp.int32), 1)
    y = plsc.bitcast(bits, jnp.float32)
    for _ in range(3):
        y = y * (1.5 - 0.5 * x * y * y)
    return y


def _sqrt(x):
    return x * _rsqrt(x)


def _log(x):
    bits = plsc.bitcast(x, jnp.int32)
    e = lax.shift_right_logical(bits, 23) - 127
    m_bits = (bits & jnp.int32(0x007FFFFF)) | jnp.int32(0x3F800000)
    m = plsc.bitcast(m_bits, jnp.float32)
    big = m > 1.4142135
    m = jnp.where(big, m * 0.5, m)
    e = jnp.where(big, e + 1, e).astype(jnp.float32)
    z = (m - 1.0) / (m + 1.0)
    z2 = z * z
    p = 2.0 * z * (1.0 + z2 * (1.0 / 3.0 + z2 * (0.2 + z2 * (1.0 / 7.0 + z2 / 9.0))))
    return p + e * _LN2


def _artanh(x):
    return 0.5 * _log((1.0 + x) / (1.0 - x))


def _tanh(x):
    t = jnp.exp(-2.0 * x)
    return (1.0 - t) / (1.0 + t)


def _clip_scale(sq):
    nrm = _sqrt(sq)
    return jnp.where(nrm >= 1.0, 1.0 / (nrm - _EPS), jnp.float32(1.0))


def kernel(u_idx, r_idx, v_idx, Eh, rvh, Wu, bs, bo):
    B = u_idx.shape[0]
    D = Eh.shape[1]
    info = plsc.get_sparse_core_info()
    nw_ = info.num_cores * info.num_subcores
    bpw = B // nw_

    mesh = plsc.VectorSubcoreMesh(core_axis_name="c", subcore_axis_name="s")

    @functools.partial(
        pl.kernel,
        out_type=jax.ShapeDtypeStruct((B,), jnp.float32),
        mesh=mesh,
        compiler_params=pltpu.CompilerParams(
            use_tc_tiling_on_sc=True, needs_layout_passes=False),
        scratch_types=[
            pltpu.VMEM((bpw,), jnp.int32),
            pltpu.VMEM((bpw,), jnp.int32),
            pltpu.VMEM((bpw,), jnp.int32),
            pltpu.VMEM((_CH, D), jnp.float32),
            pltpu.VMEM((_CH, D), jnp.float32),
            pltpu.VMEM((_CH, D), jnp.float32),
            pltpu.VMEM((_CH, D), jnp.float32),
            pltpu.VMEM((bpw,), jnp.float32),
            pltpu.VMEM((bpw,), jnp.float32),
            pltpu.VMEM((bpw,), jnp.float32),
            pltpu.SemaphoreType.DMA,
            pltpu.SemaphoreType.DMA,
        ],
    )
    def k(eh, rv, wu, bs_t, bo_t, uix, rix, vix, out,
          uix_v, rix_v, vix_v, u_v, v_v, ru_v, rg_v, bsu_v, bov_v, res_v,
          sem, bsem):
        wid = lax.axis_index("s") * info.num_cores + lax.axis_index("c")
        base = wid * bpw
        pltpu.sync_copy(uix.at[pl.ds(base, bpw)], uix_v)
        pltpu.sync_copy(rix.at[pl.ds(base, bpw)], rix_v)
        pltpu.sync_copy(vix.at[pl.ds(base, bpw)], vix_v)

        bdescs = []
        for j in range(bpw // _IDX_CHUNK):
            sl = pl.ds(j * _IDX_CHUNK, _IDX_CHUNK)
            bdescs.append(pltpu.async_copy(bs_t.at[uix_v.at[sl]], bsu_v.at[sl], bsem))
            bdescs.append(pltpu.async_copy(bo_t.at[vix_v.at[sl]], bov_v.at[sl], bsem))
        for d in bdescs:
            d.wait()

        def chunk(c, _):
            coff = pl.multiple_of(c * _CH, _CH)

            def grp(g, _):
                goff = pl.multiple_of(coff + g * 16, 16)
                boff = pl.multiple_of(g * 16, 16)
                iu = uix_v[pl.ds(goff, 16)]
                iv = vix_v[pl.ds(goff, 16)]
                ir = rix_v[pl.ds(goff, 16)]
                descs = []
                for j in range(16):
                    dst = pl.ds(boff + j, 1)
                    descs.append(pltpu.async_copy(eh.at[pl.ds(iu[j], 1)], u_v.at[dst], sem))
                    descs.append(pltpu.async_copy(eh.at[pl.ds(iv[j], 1)], v_v.at[dst], sem))
                    descs.append(pltpu.async_copy(wu.at[pl.ds(ir[j], 1)], ru_v.at[dst], sem))
                    descs.append(pltpu.async_copy(rv.at[pl.ds(ir[j], 1)], rg_v.at[dst], sem))
                for d in descs:
                    d.wait()
                return ()

            lax.fori_loop(0, _CH // 16, grp, (), unroll=False)

            def mgrp(g, _):
                boff = pl.multiple_of(g * 16, 16)
                goff = pl.multiple_of(coff + g * 16, 16)
                rows = lax.iota(jnp.int32, 16) + boff
                zero = jnp.zeros((16,), jnp.float32)
                squ = zero
                sqv = zero
                sqrg = zero
                dvr = zero
                sqp = zero
                dpv = zero
                dprg = zero
                for d in range(D):
                    col = jnp.full((16,), d, jnp.int32)
                    ud = plsc.load_gather(u_v, [rows, col])
                    vd = plsc.load_gather(v_v, [rows, col])
                    rud = plsc.load_gather(ru_v, [rows, col])
                    rgd = plsc.load_gather(rg_v, [rows, col])
                    pd = ud * rud
                    squ = squ + ud * ud
                    sqv = sqv + vd * vd
                    sqrg = sqrg + rgd * rgd
                    dvr = dvr + vd * rgd
                    sqp = sqp + pd * pd
                    dpv = dpv + pd * vd
                    dprg = dprg + pd * rgd

                su = _clip_scale(squ)
                sv = _clip_scale(sqv)
                srg = _clip_scale(sqrg)
                squ_c = squ * su * su
                nu = jnp.clip(_sqrt(squ_c), 1e-10, 1.0 - 1e-5)
                lam = (_artanh(nu) / nu) * su
                sq_uw = lam * lam * sqp
                nww = jnp.maximum(_sqrt(sq_uw), 1e-10)
                mu = (_tanh(nww) / nww) * lam
                sq_um = mu * mu * sqp
                sqx = jnp.clip(sqv * sv * sv, 0.0, 1.0 - 1e-5)
                sqy = jnp.clip(sqrg * srg * srg, 0.0, 1.0 - 1e-5)
                dot = dvr * sv * srg
                a1 = 1.0 + 2.0 * dot + sqy
                b1 = 1.0 - sqx
                c1 = 1.0 + 2.0 * dot + sqx * sqy
                av = a1 * sv / c1
                bg = b1 * srg / c1
                sq_vm = av * av * sqv + 2.0 * av * bg * dvr + bg * bg * sqrg
                dot_umvm = mu * (av * dpv + bg * dprg)
                sum_s = _clip_scale(sq_um)
                svm_s = _clip_scale(sq_vm)
                sq_um_c = sq_um * sum_s * sum_s
                sq_vm_c = sq_vm * svm_s * svm_s
                dot_c = dot_umvm * sum_s * svm_s
                sqx2 = jnp.clip(sq_um_c, 0.0, 1.0 - 1e-5)
                sqy2 = jnp.clip(sq_vm_c, 0.0, 1.0 - 1e-5)
                dot2 = -dot_c
                a2 = 1.0 + 2.0 * dot2 + sqy2
                b2 = 1.0 - sqx2
                c2 = 1.0 + 2.0 * dot2 + sqx2 * sqy2
                sq_d = a2 * a2 * sq_um_c + 2.0 * a2 * b2 * dot2 + b2 * b2 * sq_vm_c
                dn = jnp.clip(_sqrt(sq_d) / jnp.abs(c2), 1e-10, 1.0 - 1e-5)
                at = _artanh(dn)
                sqdist = 4.0 * at * at
                res = -sqdist + bsu_v[pl.ds(goff, 16)] + bov_v[pl.ds(goff, 16)]
                res_v[pl.ds(goff, 16)] = res
                return ()

            lax.fori_loop(0, _CH // 16, mgrp, (), unroll=False)
            return ()

        lax.fori_loop(0, bpw // _CH, chunk, (), unroll=False)
        pltpu.sync_copy(res_v, out.at[pl.ds(base, bpw)])

    return k(Eh, rvh, Wu, bs, bo, u_idx, r_idx, v_idx)
